# baseline (speedup 1.0000x reference)
import functools

import jax
import jax.numpy as jnp
from jax import lax
from jax.experimental import pallas as pl
from jax.experimental.pallas import tpu as pltpu

_LANE = 128
_INT_MIN = -2147483648
_MASK31 = 0x7FFFFFFF
_NEG_INF = float("-inf")


def _body(v_ref, k_ref, p_ref, o_ref, skey_ref, e_ref,
          lo_ref, hi_ref, na_ref, nb_ref, dn_ref):
    v = v_ref[...]
    b = lax.bitcast_convert_type(v, jnp.int32)
    skey = jnp.where(b >= 0, b, b ^ _MASK31)
    skey_ref[...] = skey
    m = jnp.max(v, axis=1, keepdims=True)
    maxkey = jnp.max(skey, axis=1, keepdims=True)
    e = jnp.exp(v - m)
    e_ref[...] = e
    kv = k_ref[:, :1]
    pv = p_ref[:, :1]

    vp = v.shape[1]
    csz = 98 * 128

    def _chunks():
        return [(j, min(j + csz, vp)) for j in range(0, vp, csz)]

    def _combine(parts):
        tot = parts[0]
        for q in parts[1:]:
            tot = tot + q
        return tot

    def cnt_ge(cand):
        return _combine([
            jnp.sum((skey_ref[:, a:z] >= cand).astype(jnp.int32),
                    axis=1, keepdims=True)
            for a, z in _chunks()
        ])

    zero = jnp.zeros_like(kv)
    minkey = jnp.min(skey, axis=1, keepdims=True)
    lo_ref[:, :1] = minkey
    hi_ref[:, :1] = maxkey + 1
    na_ref[:, :1] = jnp.full_like(kv, vp)
    dn_ref[:, :1] = zero

    def _cond(c):
        return c > 0

    def s1_step(c):
        lo = lo_ref[:, :1]
        hi = hi_ref[:, :1]
        nlo = na_ref[:, :1]
        done = dn_ref[:, :1] > 0
        mid = lo + lax.shift_right_logical(hi - lo, 1)
        n = cnt_ge(mid)
        take = n >= kv
        lo2 = jnp.where(done, lo, jnp.where(take, mid, lo))
        hi2 = jnp.where(done, hi, jnp.where(take, hi, mid))
        n2 = jnp.where(done, nlo, jnp.where(take, n, nlo))
        half2 = lax.shift_right_logical(hi2 - lo2, 1)
        done2 = done | (n2 == kv) | (half2 == 0)
        lo_ref[:, :1] = lo2
        hi_ref[:, :1] = hi2
        na_ref[:, :1] = n2
        dn_ref[:, :1] = done2.astype(jnp.int32)
        return jnp.sum(1 - done2.astype(jnp.int32))

    lax.while_loop(_cond, s1_step, jnp.int32(v.shape[0]))
    lo_f = lo_ref[:, :1]
    n_f = na_ref[:, :1]

    min_parts = [
        jnp.min(jnp.where(skey_ref[:, a:z] >= lo_f, skey_ref[:, a:z],
                          2147483647), axis=1, keepdims=True)
        for a, z in _chunks()
    ]
    topk_min = min_parts[0]
    for q in min_parts[1:]:
        topk_min = jnp.minimum(topk_min, q)
    tk = jnp.where(n_f == kv, topk_min, lo_f)

    s = _combine([
        jnp.sum(jnp.where(skey_ref[:, a:z] >= tk, e_ref[:, a:z], 0.0),
                axis=1, keepdims=True)
        for a, z in _chunks()
    ])
    ps = pv * s

    def mass_cnt_gt(cand):
        mparts, cparts = [], []
        for a, z in _chunks():
            ev = e_ref[:, a:z]
            eb = lax.bitcast_convert_type(ev, jnp.int32)
            hit = eb > cand
            mparts.append(jnp.sum(jnp.where(hit, ev, 0.0),
                                  axis=1, keepdims=True))
            cparts.append(jnp.sum(hit.astype(jnp.int32),
                                  axis=1, keepdims=True))
        return _combine(mparts), _combine(cparts)

    lo_ref[:, :1] = jnp.full_like(kv, -1)
    hi_ref[:, :1] = jnp.full_like(kv, 0x3F800001)
    na_ref[:, :1] = jnp.full_like(kv, vp)
    nb_ref[:, :1] = zero
    dn_ref[:, :1] = zero

    def s2_step(c):
        lo = lo_ref[:, :1]
        hi = hi_ref[:, :1]
        nlo = na_ref[:, :1]
        nhi = nb_ref[:, :1]
        done = dn_ref[:, :1] > 0
        mid = lo + lax.shift_right_logical(hi - lo, 1)
        mass, n = mass_cnt_gt(mid)
        take = mass >= ps
        lo2 = jnp.where(done, lo, jnp.where(take, mid, lo))
        hi2 = jnp.where(done, hi, jnp.where(take, hi, mid))
        nlo2 = jnp.where(done, nlo, jnp.where(take, n, nlo))
        nhi2 = jnp.where(done, nhi, jnp.where(take, nhi, n))
        half2 = lax.shift_right_logical(hi2 - lo2, 1)
        done2 = done | (nlo2 - nhi2 == 1) | (half2 == 0)
        lo_ref[:, :1] = lo2
        hi_ref[:, :1] = hi2
        na_ref[:, :1] = nlo2
        nb_ref[:, :1] = nhi2
        dn_ref[:, :1] = done2.astype(jnp.int32)
        return jnp.sum(1 - done2.astype(jnp.int32))

    lax.while_loop(_cond, s2_step, jnp.int32(v.shape[0]))
    lo2_f = lo_ref[:, :1]
    nlo2_f = na_ref[:, :1]
    nhi2_f = nb_ref[:, :1]

    u_parts = []
    for a, z in _chunks():
        eb = lax.bitcast_convert_type(e_ref[:, a:z], jnp.int32)
        u_parts.append(jnp.min(jnp.where(eb > lo2_f, eb, 2147483647),
                               axis=1, keepdims=True))
    u = u_parts[0]
    for q in u_parts[1:]:
        u = jnp.minimum(u, q)
    mp = jnp.where(nlo2_f - nhi2_f == 1, u - 1, lo2_f)

    sk = skey_ref[...]
    eb = lax.bitcast_convert_type(e_ref[...], jnp.int32)
    keep = (sk >= tk) & ((eb > mp) | (sk == maxkey))
    o_ref[...] = jnp.where(keep, v, _NEG_INF)


@functools.partial(jax.jit, static_argnames=())
def kernel(logits, k, p):
    bsz, vocab = logits.shape
    vp = pl.cdiv(vocab, _LANE) * _LANE
    rblk = 16
    logits = logits.astype(jnp.float32)
    if vp != vocab:
        pad = jnp.full((bsz, vp - vocab), _NEG_INF, jnp.float32)
        lp = jnp.concatenate([logits, pad], axis=1)
    else:
        lp = logits
    kb = jnp.broadcast_to(
        jnp.clip(k.astype(jnp.int32), 1, vocab)[:, None], (bsz, _LANE))
    pb = jnp.broadcast_to(p.astype(jnp.float32)[:, None], (bsz, _LANE))
    out = pl.pallas_call(
        _body,
        grid=(bsz // rblk,),
        in_specs=[
            pl.BlockSpec((rblk, vp), lambda i: (i, 0)),
            pl.BlockSpec((rblk, _LANE), lambda i: (i, 0)),
            pl.BlockSpec((rblk, _LANE), lambda i: (i, 0)),
        ],
        out_specs=pl.BlockSpec((rblk, vp), lambda i: (i, 0)),
        out_shape=jax.ShapeDtypeStruct((bsz, vp), jnp.float32),
        scratch_shapes=[
            pltpu.VMEM((rblk, vp), jnp.int32),
            pltpu.VMEM((rblk, vp), jnp.float32),
            pltpu.VMEM((rblk, _LANE), jnp.int32),
            pltpu.VMEM((rblk, _LANE), jnp.int32),
            pltpu.VMEM((rblk, _LANE), jnp.int32),
            pltpu.VMEM((rblk, _LANE), jnp.int32),
            pltpu.VMEM((rblk, _LANE), jnp.int32),
        ],
    )(lp, kb, pb)
    return out[:, :vocab]


# speedup vs baseline: 1.1304x; 1.1304x over previous
import functools

import jax
import jax.numpy as jnp
from jax import lax
from jax.experimental import pallas as pl
from jax.experimental.pallas import tpu as pltpu

_LANE = 128
_INT_MIN = -2147483648
_MASK31 = 0x7FFFFFFF
_NEG_INF = float("-inf")


def _body(v_ref, k_ref, p_ref, o_ref, skey_ref, e_ref):
    v = v_ref[...]
    b = lax.bitcast_convert_type(v, jnp.int32)
    skey = jnp.where(b >= 0, b, b ^ _MASK31)
    skey_ref[...] = skey
    m = jnp.max(v, axis=1, keepdims=True)
    maxkey = jnp.max(skey, axis=1, keepdims=True)
    e = jnp.exp(v - m)
    e_ref[...] = e
    kv = k_ref[:, :1]
    pv = p_ref[:, :1]

    vp = v.shape[1]
    csz = 49 * 128

    def _rowsum(x):
        parts = [
            jnp.sum(x[:, j:min(j + csz, vp)], axis=1, keepdims=True)
            for j in range(0, vp, csz)
        ]
        tot = parts[0]
        for q in parts[1:]:
            tot = tot + q
        return tot

    def cnt_ge(cand):
        return _rowsum((skey_ref[...] >= cand).astype(jnp.int32))

    zero = jnp.zeros_like(kv)
    base = jnp.where(cnt_ge(zero) >= kv, 0, _INT_MIN)

    def step1(i, rem):
        bit = jnp.left_shift(jnp.int32(1), 30 - i)
        cand = base + (rem | bit)
        return jnp.where(cnt_ge(cand) >= kv, rem | bit, rem)

    tk = base + lax.fori_loop(0, 31, step1, zero)

    s = _rowsum(jnp.where(skey_ref[...] >= tk, e_ref[...], 0.0))
    ps = pv * s

    def mass_gt(cand):
        ev = e_ref[...]
        eb = lax.bitcast_convert_type(ev, jnp.int32)
        return _rowsum(jnp.where(eb > cand, ev, 0.0))

    def step2(i, rem):
        bit = jnp.left_shift(jnp.int32(1), 29 - i)
        cand = rem | bit
        return jnp.where(mass_gt(cand) >= ps, rem | bit, rem)

    mp = lax.fori_loop(0, 30, step2, zero)

    sk = skey_ref[...]
    eb = lax.bitcast_convert_type(e_ref[...], jnp.int32)
    keep = (sk >= tk) & ((eb > mp) | (sk == maxkey))
    o_ref[...] = jnp.where(keep, v, _NEG_INF)


@functools.partial(jax.jit, static_argnames=())
def kernel(logits, k, p):
    bsz, vocab = logits.shape
    vp = pl.cdiv(vocab, _LANE) * _LANE
    rblk = 16
    logits = logits.astype(jnp.float32)
    if vp != vocab:
        pad = jnp.full((bsz, vp - vocab), _NEG_INF, jnp.float32)
        lp = jnp.concatenate([logits, pad], axis=1)
    else:
        lp = logits
    kb = jnp.broadcast_to(
        jnp.clip(k.astype(jnp.int32), 1, vocab)[:, None], (bsz, _LANE))
    pb = jnp.broadcast_to(p.astype(jnp.float32)[:, None], (bsz, _LANE))
    out = pl.pallas_call(
        _body,
        grid=(bsz // rblk,),
        in_specs=[
            pl.BlockSpec((rblk, vp), lambda i: (i, 0)),
            pl.BlockSpec((rblk, _LANE), lambda i: (i, 0)),
            pl.BlockSpec((rblk, _LANE), lambda i: (i, 0)),
        ],
        out_specs=pl.BlockSpec((rblk, vp), lambda i: (i, 0)),
        out_shape=jax.ShapeDtypeStruct((bsz, vp), jnp.float32),
        scratch_shapes=[
            pltpu.VMEM((rblk, vp), jnp.int32),
            pltpu.VMEM((rblk, vp), jnp.float32),
        ],
    )(lp, kb, pb)
    return out[:, :vocab]


# speedup vs baseline: 1.1458x; 1.0137x over previous
import functools

import jax
import jax.numpy as jnp
from jax import lax
from jax.experimental import pallas as pl
from jax.experimental.pallas import tpu as pltpu

_LANE = 128
_INT_MIN = -2147483648
_MASK31 = 0x7FFFFFFF
_NEG_INF = float("-inf")


def _body(v_ref, k_ref, p_ref, o_ref, e_ref):
    v = v_ref[...]
    m = jnp.max(v, axis=1, keepdims=True)
    e = jnp.exp(v - m)
    e_ref[...] = e
    kv = k_ref[:, :1]
    pv = p_ref[:, :1]

    vp = v.shape[1]
    csz = 25 * 128

    def _rowsum(x):
        parts = [
            jnp.sum(x[:, j:min(j + csz, vp)], axis=1, keepdims=True)
            for j in range(0, vp, csz)
        ]
        tot = parts[0]
        for q in parts[1:]:
            tot = tot + q
        return tot

    def _key_to_f32(ck):
        ck = jnp.where(ck >= 0, ck, jnp.maximum(ck, -2139095041))
        fb = jnp.where(ck >= 0, ck, ck ^ _MASK31)
        return lax.bitcast_convert_type(fb, jnp.float32)

    def cnt_ge(cand):
        cf = _key_to_f32(cand)
        return _rowsum((v_ref[...] >= cf).astype(jnp.int32))

    zero = jnp.zeros_like(kv)
    base = jnp.where(cnt_ge(zero) >= kv, 0, _INT_MIN)

    def step1(i, rem):
        bit = jnp.left_shift(jnp.int32(1), 30 - i)
        cand = base + (rem | bit)
        return jnp.where(cnt_ge(cand) >= kv, rem | bit, rem)

    tkf = _key_to_f32(base + lax.fori_loop(0, 31, step1, zero))

    s = _rowsum(jnp.where(v_ref[...] >= tkf, e_ref[...], 0.0))
    ps = pv * s

    def mass_gt(cand):
        ev = e_ref[...]
        eb = lax.bitcast_convert_type(ev, jnp.int32)
        return _rowsum(jnp.where(eb > cand, ev, 0.0))

    def step2(i, rem):
        bit = jnp.left_shift(jnp.int32(1), 29 - i)
        cand = rem | bit
        return jnp.where(mass_gt(cand) >= ps, rem | bit, rem)

    mp = lax.fori_loop(0, 30, step2, zero)

    eb = lax.bitcast_convert_type(e_ref[...], jnp.int32)
    keep = (v >= tkf) & ((eb > mp) | (v == m))
    o_ref[...] = jnp.where(keep, v, _NEG_INF)


@functools.partial(jax.jit, static_argnames=())
def kernel(logits, k, p):
    bsz, vocab = logits.shape
    vp = pl.cdiv(vocab, _LANE) * _LANE
    rblk = 16
    logits = logits.astype(jnp.float32)
    if vp != vocab:
        pad = jnp.full((bsz, vp - vocab), _NEG_INF, jnp.float32)
        lp = jnp.concatenate([logits, pad], axis=1)
    else:
        lp = logits
    kb = jnp.broadcast_to(
        jnp.clip(k.astype(jnp.int32), 1, vocab)[:, None], (bsz, _LANE))
    pb = jnp.broadcast_to(p.astype(jnp.float32)[:, None], (bsz, _LANE))
    out = pl.pallas_call(
        _body,
        grid=(bsz // rblk,),
        in_specs=[
            pl.BlockSpec((rblk, vp), lambda i: (i, 0)),
            pl.BlockSpec((rblk, _LANE), lambda i: (i, 0)),
            pl.BlockSpec((rblk, _LANE), lambda i: (i, 0)),
        ],
        out_specs=pl.BlockSpec((rblk, vp), lambda i: (i, 0)),
        out_shape=jax.ShapeDtypeStruct((bsz, vp), jnp.float32),
        scratch_shapes=[
            pltpu.VMEM((rblk, vp), jnp.float32),
        ],
    )(lp, kb, pb)
    return out[:, :vocab]
